# baseline (speedup 1.0000x reference)
import functools

import jax
import jax.numpy as jnp
from jax import lax
from jax.experimental import pallas as pl
from jax.experimental.pallas import tpu as pltpu
from jax.experimental.pallas import tpu_sc as plsc

_BATCH = 16384
_FIELDS = 26
_L2 = 1e-06

_NC = 2
_NS = 16
_NW = _NC * _NS
_BPW = _BATCH // _NW
_CHUNK = _FIELDS * _BPW
_LANES = 16
_NSTREAM = 4
_RPC = _BPW // _NSTREAM
_QS = _CHUNK // _NSTREAM


def _sc_gather_sum(idx_ref, w_ref, xw_ref, idx_v, vals_v, acc_v, *sems):
  wid = lax.axis_index("s") * _NC + lax.axis_index("c")
  copies = []
  for k in range(_NSTREAM):
    pltpu.sync_copy(idx_ref.at[wid, pl.ds(k * _QS, _QS)],
                    idx_v.at[pl.ds(k * _QS, _QS)])
    copies.append(
        pltpu.async_copy(
            w_ref.at[idx_v.at[pl.ds(k * _QS, _QS)]],
            vals_v.at[pl.ds(k * _QS, _QS)],
            sems[k],
        ))
  for k in range(_NSTREAM):
    copies[k].wait()
    for g in range(_RPC // _LANES):
      s = k * _QS + g * _LANES
      acc = vals_v[pl.ds(s, _LANES)]
      for f in range(1, _FIELDS):
        acc = acc + vals_v[pl.ds(s + f * _RPC, _LANES)]
      acc_v[pl.ds(k * _RPC + g * _LANES, _LANES)] = acc
  pltpu.sync_copy(acc_v, xw_ref.at[pl.ds(wid * _BPW, _BPW)])


@jax.jit
def _sc_xw(idx_arranged, w1d):
  mesh = plsc.VectorSubcoreMesh(core_axis_name="c", subcore_axis_name="s")
  return pl.kernel(
      _sc_gather_sum,
      out_type=jax.ShapeDtypeStruct((_BATCH,), jnp.float32),
      mesh=mesh,
      scratch_types=[
          pltpu.VMEM((_CHUNK,), jnp.int32),
          pltpu.VMEM((_CHUNK,), jnp.float32),
          pltpu.VMEM((_BPW,), jnp.float32),
      ] + [pltpu.SemaphoreType.DMA] * _NSTREAM,
  )(idx_arranged, w1d)


def _tc_head(xw_ref, y_ref, b_ref, yprob_ref, loss_ref):
  xw = xw_ref[...]
  logits = xw + b_ref[0]
  yprob_ref[...] = 1.0 / (1.0 + jnp.exp(-logits))
  bce = (jnp.maximum(logits, 0.0) - logits * y_ref[...]
         + jnp.log1p(jnp.exp(-jnp.abs(logits))))
  loss_ref[0] = (jnp.sum(bce) / _BATCH) + _L2 * 0.5 * jnp.sum(xw * xw)


@jax.jit
def _tc_loss(xw, y, b):
  yprob, loss = pl.pallas_call(
      _tc_head,
      out_shape=(
          jax.ShapeDtypeStruct((128, 128), jnp.float32),
          jax.ShapeDtypeStruct((1,), jnp.float32),
      ),
      in_specs=[
          pl.BlockSpec(memory_space=pltpu.VMEM),
          pl.BlockSpec(memory_space=pltpu.VMEM),
          pl.BlockSpec(memory_space=pltpu.SMEM),
      ],
      out_specs=(
          pl.BlockSpec(memory_space=pltpu.VMEM),
          pl.BlockSpec(memory_space=pltpu.SMEM),
      ),
  )(xw.reshape(128, 128), y.reshape(128, 128), b)
  return yprob.reshape(-1), loss[0]


def kernel(indices, y, w, b):
  idx = indices.astype(jnp.int32)
  idx_arranged = (
      idx.reshape(_NW, _NSTREAM, _RPC, _FIELDS)
      .transpose(0, 1, 3, 2)
      .reshape(_NW, _CHUNK)
  )
  xw = _sc_xw(idx_arranged, w.reshape(-1))
  return _tc_loss(xw, y, b)


# speedup vs baseline: 1.0038x; 1.0038x over previous
import functools

import jax
import jax.numpy as jnp
from jax import lax
from jax.experimental import pallas as pl
from jax.experimental.pallas import tpu as pltpu
from jax.experimental.pallas import tpu_sc as plsc

_BATCH = 16384
_FIELDS = 26
_L2 = 1e-06

_NC = 2
_NS = 16
_NW = _NC * _NS
_BPW = _BATCH // _NW
_CHUNK = _FIELDS * _BPW
_LANES = 16


def _sc_gather_sum(idx_ref, w_ref, xw_ref, idx_v, vals_v, acc_v, sem):
  wid = lax.axis_index("s") * _NC + lax.axis_index("c")
  pltpu.sync_copy(idx_ref.at[wid], idx_v)
  pltpu.async_copy(w_ref.at[idx_v], vals_v, sem).wait()
  for g in range(_BPW // _LANES):
    acc = vals_v[pl.ds(g * _LANES, _LANES)]
    for f in range(1, _FIELDS):
      acc = acc + vals_v[pl.ds(f * _BPW + g * _LANES, _LANES)]
    acc_v[pl.ds(g * _LANES, _LANES)] = acc
  pltpu.sync_copy(acc_v, xw_ref.at[pl.ds(wid * _BPW, _BPW)])


@jax.jit
def _sc_xw(idx_arranged, w1d):
  mesh = plsc.VectorSubcoreMesh(core_axis_name="c", subcore_axis_name="s")
  return pl.kernel(
      _sc_gather_sum,
      out_type=jax.ShapeDtypeStruct((_BATCH,), jnp.float32),
      mesh=mesh,
      scratch_types=[
          pltpu.VMEM((_CHUNK,), jnp.int32),
          pltpu.VMEM((_CHUNK,), jnp.float32),
          pltpu.VMEM((_BPW,), jnp.float32),
          pltpu.SemaphoreType.DMA,
      ],
  )(idx_arranged, w1d)


def _tc_head(xw_ref, y_ref, b_ref, yprob_ref, loss_ref):
  xw = xw_ref[...]
  logits = xw + b_ref[0]
  yprob_ref[...] = 1.0 / (1.0 + jnp.exp(-logits))
  bce = (jnp.maximum(logits, 0.0) - logits * y_ref[...]
         + jnp.log1p(jnp.exp(-jnp.abs(logits))))
  loss_ref[0] = (jnp.sum(bce) / _BATCH) + _L2 * 0.5 * jnp.sum(xw * xw)


@jax.jit
def _tc_loss(xw, y, b):
  yprob, loss = pl.pallas_call(
      _tc_head,
      out_shape=(
          jax.ShapeDtypeStruct((128, 128), jnp.float32),
          jax.ShapeDtypeStruct((1,), jnp.float32),
      ),
      in_specs=[
          pl.BlockSpec(memory_space=pltpu.VMEM),
          pl.BlockSpec(memory_space=pltpu.VMEM),
          pl.BlockSpec(memory_space=pltpu.SMEM),
      ],
      out_specs=(
          pl.BlockSpec(memory_space=pltpu.VMEM),
          pl.BlockSpec(memory_space=pltpu.SMEM),
      ),
  )(xw.reshape(128, 128), y.reshape(128, 128), b)
  return yprob.reshape(-1), loss[0]


def kernel(indices, y, w, b):
  idx = indices.astype(jnp.int32)
  idx_arranged = (
      idx.reshape(_NW, _BPW, _FIELDS).transpose(0, 2, 1).reshape(_NW, _CHUNK)
  )
  xw = _sc_xw(idx_arranged, w.reshape(-1))
  return _tc_loss(xw, y, b)
